# baseline (speedup 1.0000x reference)
import functools

import jax
import jax.numpy as jnp
from jax import lax
from jax.experimental import pallas as pl
from jax.experimental.pallas import tpu as pltpu
from jax.experimental.pallas import tpu_sc as plsc

_NENTITY = 1000000
_EMBED_DIM = 16
_N_VEC = 4
_BATCH = 16384
_ROW = _EMBED_DIM * _N_VEC
_PAIR = 2 * _ROW

_SLAB = 256
_NSLAB = 999936 // _SLAB
_TAIL = _NENTITY - 999936

_NW = 32
_RPW = _BATCH // _NW
_CHUNK = 128
_NCHUNK = _RPW // _CHUNK
_GROUPS = _RPW * _N_VEC // 16


def _rsqrt16(x):
    i = plsc.bitcast(x, jnp.int32)
    i = 0x5F3759DF - (i >> 1)
    y = plsc.bitcast(i, jnp.float32)
    xh = x * 0.5
    for _ in range(3):
        y = y * (1.5 - xh * y * y)
    return y


@functools.partial(
    pl.kernel,
    mesh=plsc.VectorSubcoreMesh(core_axis_name="c", subcore_axis_name="s"),
    out_type=jax.ShapeDtypeStruct((_NENTITY // 2, _PAIR), jnp.float32),
    scratch_types=[
        pltpu.VMEM((2, _ROW, _SLAB), jnp.float32),
        pltpu.VMEM((2, _SLAB // 2, _PAIR), jnp.float32),
        pltpu.VMEM((_ROW, _TAIL), jnp.float32),
        pltpu.SemaphoreType.DMA,
        pltpu.SemaphoreType.DMA,
    ],
    compiler_params=pltpu.CompilerParams(needs_layout_passes=False),
)
def _transpose(tablet_hbm, tail_hbm, pairs_hbm, slab_v, outb_v, tail_v, insem, outsem):
    wid = lax.axis_index("s") * 2 + lax.axis_index("c")
    cnt = (_NSLAB - wid + 31) // 32

    lane = lax.iota(jnp.int32, 16)
    half = lane >> 1
    parity64 = (lane & 1) * _ROW

    def transpose_slab(slot, width):
        ob = outb_v.at[slot]
        for g in range(width // 16):
            r0 = g * 16
            rowv = half + (r0 >> 1)
            for c in range(_ROW):
                v = slab_v[slot, c, pl.ds(r0, 16)]
                plsc.store_scatter(ob, [rowv, parity64 + c], v)

    def fire_in(i):
        slab = wid + i * 32
        return pltpu.async_copy(
            tablet_hbm.at[:, pl.ds(slab * _SLAB, _SLAB)],
            slab_v.at[lax.rem(i, 2)],
            insem,
        )

    fire_in(0)

    def slab_body(i, carry):
        slot = lax.rem(i, 2)

        @pl.when(i + 1 < cnt)
        def _():
            fire_in(i + 1)

        pltpu.make_async_copy(
            tablet_hbm.at[:, pl.ds(0, _SLAB)], slab_v.at[slot], insem
        ).wait()

        @pl.when(i >= 2)
        def _():
            pltpu.make_async_copy(
                outb_v.at[slot], pairs_hbm.at[pl.ds(0, _SLAB // 2)], outsem
            ).wait()

        transpose_slab(slot, _SLAB)
        slab = wid + i * 32
        pltpu.async_copy(
            outb_v.at[slot],
            pairs_hbm.at[pl.ds(slab * (_SLAB // 2), _SLAB // 2)],
            outsem,
        )
        return carry

    lax.fori_loop(0, cnt, slab_body, 0)

    for _ in range(2):
        pltpu.make_async_copy(
            outb_v.at[0], pairs_hbm.at[pl.ds(0, _SLAB // 2)], outsem
        ).wait()

    @pl.when(wid == 0)
    def _():
        pltpu.sync_copy(tail_hbm, tail_v)
        for g in range(_TAIL // 16):
            r0 = g * 16
            rowv = half + (r0 >> 1)
            for c in range(_ROW):
                v = tail_v[c, pl.ds(r0, 16)]
                plsc.store_scatter(outb_v.at[0], [rowv, parity64 + c], v)
        pltpu.sync_copy(
            outb_v.at[0, pl.ds(0, _TAIL // 2)],
            pairs_hbm.at[pl.ds(999936 // 2, _TAIL // 2)],
        )


@functools.partial(
    pl.kernel,
    mesh=plsc.VectorSubcoreMesh(core_axis_name="c", subcore_axis_name="s"),
    out_type=jax.ShapeDtypeStruct((_ROW, _BATCH), jnp.float32),
    scratch_types=[
        pltpu.VMEM((_NCHUNK, _CHUNK), jnp.int32),
        pltpu.VMEM((_NCHUNK, _CHUNK), jnp.int32),
        pltpu.VMEM((_RPW, _PAIR), jnp.float32),
        pltpu.VMEM((_ROW, _RPW), jnp.float32),
        pltpu.SemaphoreType.DMA,
    ],
    compiler_params=pltpu.CompilerParams(needs_layout_passes=False),
)
def _gathernorm(idx_hbm, pairs_hbm, out_hbm, idx_v, gidx_v, rows_v, out_v, sem):
    wid = lax.axis_index("s") * 2 + lax.axis_index("c")
    base = wid * _RPW

    pltpu.sync_copy(idx_hbm.at[pl.ds(wid * _NCHUNK, _NCHUNK)], idx_v)
    for j in range(_NCHUNK):
        for t in range(_CHUNK // 16):
            gidx_v[j, pl.ds(t * 16, 16)] = idx_v[j, pl.ds(t * 16, 16)] >> 1
    copies = [
        pltpu.async_copy(
            pairs_hbm.at[gidx_v.at[j]],
            rows_v.at[pl.ds(j * _CHUNK, _CHUNK)],
            sem,
        )
        for j in range(_NCHUNK)
    ]
    for cp in copies:
        cp.wait()

    lane = lax.iota(jnp.int32, 16)
    rowoff = lane >> 2
    coloff = (lane & 3) * 16

    def group_body(grp, carry):
        row_idx = rowoff + grp * 4
        orig = plsc.load_gather(idx_v, [row_idx >> 7, row_idx & 127])
        src_col = coloff + (orig & 1) * _ROW
        vs = []
        acc = None
        for p in range(16):
            v = plsc.load_gather(rows_v, [row_idx, src_col + p])
            vs.append(v)
            sq = v * v
            acc = sq if acc is None else acc + sq
        scale = _rsqrt16(acc)
        for p in range(16):
            plsc.store_scatter(out_v, [coloff + p, row_idx], vs[p] * scale)
        return carry

    lax.fori_loop(0, _GROUPS, group_body, 0)

    pltpu.sync_copy(out_v, out_hbm.at[:, pl.ds(base, _RPW)])


def kernel(indices, table):
    idx = indices.astype(jnp.int32).reshape(_BATCH // _CHUNK, _CHUNK)
    tablet = table.T
    tail = lax.slice(tablet, (0, 999936), (_ROW, _NENTITY))
    pairs = _transpose(tablet, tail)
    out = _gathernorm(idx, pairs)
    return out.reshape(_N_VEC, _EMBED_DIM, _BATCH).transpose(2, 0, 1)


# speedup vs baseline: 3.3739x; 3.3739x over previous
import functools

import jax
import jax.numpy as jnp
from jax import lax
from jax.experimental import pallas as pl
from jax.experimental.pallas import tpu as pltpu
from jax.experimental.pallas import tpu_sc as plsc

_NENTITY = 1000000
_EMBED_DIM = 16
_N_VEC = 4
_BATCH = 16384
_ROW = _EMBED_DIM * _N_VEC

_NW = 32
_RPW = _BATCH // _NW
_GROUPS = _RPW * _N_VEC // 16


def _rsqrt16(x):
    i = plsc.bitcast(x, jnp.int32)
    i = 0x5F3759DF - (i >> 1)
    y = plsc.bitcast(i, jnp.float32)
    xh = x * 0.5
    for _ in range(3):
        y = y * (1.5 - xh * y * y)
    return y


@functools.partial(
    pl.kernel,
    mesh=plsc.VectorSubcoreMesh(core_axis_name="c", subcore_axis_name="s"),
    out_type=jax.ShapeDtypeStruct((_BATCH, _ROW), jnp.float32),
    scratch_types=[
        pltpu.VMEM((_RPW,), jnp.int32),
        pltpu.VMEM((_RPW, _ROW), jnp.float32),
        pltpu.SemaphoreType.DMA,
    ],
    compiler_params=pltpu.CompilerParams(needs_layout_passes=False),
)
def _encode(idx_hbm, table_hbm, out_hbm, idx_v, rows_v, sem):
    wid = lax.axis_index("s") * 2 + lax.axis_index("c")
    base = wid * _RPW

    pltpu.sync_copy(idx_hbm.at[pl.ds(base, _RPW)], idx_v)

    def fire_block(t, carry):
        iv = idx_v[pl.ds(t * 16, 16)]
        r0 = t * 16
        for j in range(16):
            s = iv[j]
            pltpu.async_copy(
                table_hbm.at[pl.ds(s, 1)], rows_v.at[pl.ds(r0 + j, 1)], sem
            )
        return carry

    lax.fori_loop(0, _RPW // 16, fire_block, 0)

    pltpu.make_async_copy(
        table_hbm.at[pl.ds(0, _RPW)], rows_v, sem
    ).wait()

    lane = lax.iota(jnp.int32, 16)
    rowoff = lane >> 2
    coloff = (lane & 3) * 16

    def group_body(grp, carry):
        row_idx = rowoff + grp * 4
        vs = []
        acc = None
        for p in range(16):
            v = plsc.load_gather(rows_v, [row_idx, coloff + p])
            vs.append(v)
            sq = v * v
            acc = sq if acc is None else acc + sq
        scale = _rsqrt16(acc)
        for p in range(16):
            plsc.store_scatter(rows_v, [row_idx, coloff + p], vs[p] * scale)
        return carry

    lax.fori_loop(0, _GROUPS, group_body, 0)

    pltpu.sync_copy(rows_v, out_hbm.at[pl.ds(base, _RPW)])


def kernel(indices, table):
    idx = indices.astype(jnp.int32)
    out = _encode(idx, table)
    return out.reshape(_BATCH, _N_VEC, _EMBED_DIM)
